# baseline (speedup 1.0000x reference)
import jax
import jax.numpy as jnp
from jax import lax
from jax.experimental import pallas as pl
from jax.experimental.pallas import tpu as pltpu
from jax.experimental.pallas import tpu_sc as plsc

_F = 26
_N = 1000
_R = 16
_B = 16384
_L = 16
_G = _B // _L


def _sc_body(x_hbm, y_hbm, scf_hbm, std_hbm, out_hbm,
             table_v, std_v, x_v, y_v, o_v):
    c = lax.axis_index("c")
    s = lax.axis_index("s")
    f = s * 2 + c

    @pl.when(f < _F)
    def _():
        pltpu.sync_copy(scf_hbm.at[f], table_v)
        pltpu.sync_copy(std_hbm.at[f], std_v)
        pltpu.sync_copy(x_hbm.at[f], x_v)
        pltpu.sync_copy(y_hbm.at[f], y_v)

        @plsc.parallel_loop(0, _N * _R // _L, unroll=4)
        def _square(i):
            row = table_v[pl.ds(i * _L, _L)]
            table_v[pl.ds(i * _L, _L)] = row * row

        @plsc.parallel_loop(0, _G, unroll=2)
        def _group(g):
            base = g * _L
            xv = x_v[pl.ds(base, _L)]
            yv = y_v[pl.ds(base, _L)]
            acc = [jnp.zeros((_L,), jnp.float32) for _ in range(4)]
            for r in range(_R):
                ax = plsc.load_gather(table_v, [xv + r * _N])
                ay = plsc.load_gather(table_v, [yv + r * _N])
                acc[r % 4] = acc[r % 4] + ax * ay
            sx = plsc.load_gather(std_v, [xv])
            total = (acc[0] + acc[1]) + (acc[2] + acc[3])
            total = jnp.where(xv == yv, total + sx * sx, total)
            o_v[pl.ds(base, _L)] = total

        pltpu.sync_copy(o_v, out_hbm.at[f])


@jax.jit
def kernel(x, y, sqrt_covar_factor, std):
    xt = x.astype(jnp.int32).reshape(_F, _B)
    yt = y.astype(jnp.int32).reshape(_F, _B)
    scf_flat = sqrt_covar_factor.reshape(_F, _R * _N)
    mesh = plsc.VectorSubcoreMesh(core_axis_name="c", subcore_axis_name="s")
    out = pl.kernel(
        _sc_body,
        out_type=jax.ShapeDtypeStruct((_F, _B), jnp.float32),
        mesh=mesh,
        compiler_params=pltpu.CompilerParams(needs_layout_passes=False),
        scratch_types=[
            pltpu.VMEM((_R * _N,), jnp.float32),
            pltpu.VMEM((_N,), jnp.float32),
            pltpu.VMEM((_B,), jnp.int32),
            pltpu.VMEM((_B,), jnp.int32),
            pltpu.VMEM((_B,), jnp.float32),
        ],
    )(xt, yt, scf_flat, std)
    return out.reshape(_B, _F)


# speedup vs baseline: 3.0402x; 3.0402x over previous
import jax
import jax.numpy as jnp
from jax import lax
from jax.experimental import pallas as pl
from jax.experimental.pallas import tpu as pltpu
from jax.experimental.pallas import tpu_sc as plsc

_F = 26
_N = 1000
_R = 16
_B = 16384
_L = 16
_G = _B // _L
_R2 = _R // 2
_NCHUNK = (_N + _L - 1) // _L


def _sc_body(x_hbm, y_hbm, scf_hbm, std_hbm, out_hbm,
             raw_v, pk_v, std_v, x_v, y_v, o_v):
    c = lax.axis_index("c")
    s = lax.axis_index("s")
    f = s * 2 + c

    @pl.when(f < _F)
    def _():
        pltpu.sync_copy(scf_hbm.at[f], raw_v)
        pltpu.sync_copy(std_hbm.at[f], std_v)
        pltpu.sync_copy(x_hbm.at[f], x_v)
        pltpu.sync_copy(y_hbm.at[f], y_v)

        @plsc.parallel_loop(0, _R2 * _NCHUNK, unroll=2)
        def _prep(i):
            j = i // _NCHUNK
            k = i % _NCHUNK
            off = jnp.minimum(k * _L, _N - _L)
            a = raw_v[pl.ds((2 * j) * _N + off, _L)]
            b = raw_v[pl.ds((2 * j + 1) * _N + off, _L)]
            packed = plsc.pack(a * a, b * b, format=plsc.PackFormat.INTERLEAVED)
            pk_v[pl.ds(j * _N + off, _L)] = plsc.bitcast(packed, jnp.float32)

        @plsc.parallel_loop(0, _G, unroll=2)
        def _group(g):
            base = g * _L
            xv = x_v[pl.ds(base, _L)]
            yv = y_v[pl.ds(base, _L)]
            acc = jnp.zeros((2 * _L,), jnp.bfloat16)
            for j in range(_R2):
                gx = plsc.load_gather(pk_v, [xv + j * _N])
                gy = plsc.load_gather(pk_v, [yv + j * _N])
                bx = plsc.bitcast(gx, jnp.bfloat16)
                by = plsc.bitcast(gy, jnp.bfloat16)
                acc = acc + bx * by
            pe, po = plsc.unpack(acc, format=plsc.PackFormat.INTERLEAVED)
            total = pe + po
            sx = plsc.load_gather(std_v, [xv])
            total = jnp.where(xv == yv, total + sx * sx, total)
            o_v[pl.ds(base, _L)] = total

        pltpu.sync_copy(o_v, out_hbm.at[f])


@jax.jit
def kernel(x, y, sqrt_covar_factor, std):
    xt = x.astype(jnp.int32).T
    yt = y.astype(jnp.int32).T
    scf_flat = sqrt_covar_factor.transpose(0, 2, 1).reshape(_F, _R * _N)
    mesh = plsc.VectorSubcoreMesh(core_axis_name="c", subcore_axis_name="s")
    out = pl.kernel(
        _sc_body,
        out_type=jax.ShapeDtypeStruct((_F, _B), jnp.float32),
        mesh=mesh,
        compiler_params=pltpu.CompilerParams(needs_layout_passes=False),
        scratch_types=[
            pltpu.VMEM((_R * _N,), jnp.float32),
            pltpu.VMEM((_R2 * _N,), jnp.float32),
            pltpu.VMEM((_N,), jnp.float32),
            pltpu.VMEM((_B,), jnp.int32),
            pltpu.VMEM((_B,), jnp.int32),
            pltpu.VMEM((_B,), jnp.float32),
        ],
    )(xt, yt, scf_flat, std)
    return out.T
